# initial kernel scaffold (unmeasured)
import jax
import jax.numpy as jnp
from jax import lax
from jax.experimental import pallas as pl
from jax.experimental.pallas import tpu as pltpu


def kernel(
    x,
):
    def body(*refs):
        pass

    out_shape = jax.ShapeDtypeStruct(..., jnp.float32)
    return pl.pallas_call(body, out_shape=out_shape)(...)


# baseline (device time: 602722 ns/iter reference)
import functools

import jax
import jax.numpy as jnp
from jax import lax
from jax.experimental import pallas as pl
from jax.experimental.pallas import tpu as pltpu

N_Z = 4


def kernel(x):
    m, n = x.shape
    ch = m // N_Z

    def body(x_hbm, out_ref, comm_ref, copy_sem, send_sems, recv_sems):
        my_x = lax.axis_index("x")
        my_y = lax.axis_index("y")
        my_z = lax.axis_index("z")
        right = (my_z + 1) % N_Z
        left = (my_z + N_Z - 1) % N_Z

        cp = pltpu.make_async_copy(x_hbm, out_ref, copy_sem)
        cp.start()
        cp.wait()

        barrier_sem = pltpu.get_barrier_semaphore()
        for nbr in (left, right):
            pl.semaphore_signal(
                barrier_sem, inc=1,
                device_id=(my_x, my_y, nbr),
                device_id_type=pl.DeviceIdType.MESH,
            )
        pl.semaphore_wait(barrier_sem, 2)

        for s in range(N_Z - 1):
            send_idx = (my_z + N_Z - s) % N_Z
            recv_idx = (my_z + N_Z - s - 1) % N_Z
            rdma = pltpu.make_async_remote_copy(
                src_ref=out_ref.at[pl.ds(send_idx * ch, ch)],
                dst_ref=comm_ref.at[s],
                send_sem=send_sems.at[s],
                recv_sem=recv_sems.at[s],
                device_id=(my_x, my_y, right),
                device_id_type=pl.DeviceIdType.MESH,
            )
            rdma.start()
            rdma.wait()
            out_ref[pl.ds(recv_idx * ch, ch), :] = (
                out_ref[pl.ds(recv_idx * ch, ch), :] + comm_ref[s, :, :]
            )

        for t in range(N_Z - 1):
            a = (my_z + N_Z + 1 - t) % N_Z
            rdma = pltpu.make_async_remote_copy(
                src_ref=out_ref.at[pl.ds(a * ch, ch)],
                dst_ref=out_ref.at[pl.ds(a * ch, ch)],
                send_sem=send_sems.at[N_Z - 1 + t],
                recv_sem=recv_sems.at[N_Z - 1 + t],
                device_id=(my_x, my_y, right),
                device_id_type=pl.DeviceIdType.MESH,
            )
            rdma.start()
            rdma.wait()

        @functools.partial(
            pl.run_scoped, exit_sem=pltpu.SemaphoreType.REGULAR
        )
        def _(exit_sem):
            for nbr in (left, right):
                pl.semaphore_signal(
                    exit_sem, inc=1,
                    device_id=(my_x, my_y, nbr),
                    device_id_type=pl.DeviceIdType.MESH,
                )
            pl.semaphore_wait(exit_sem, 2)

    return pl.pallas_call(
        body,
        out_shape=jax.ShapeDtypeStruct((m, n), x.dtype),
        in_specs=[pl.BlockSpec(memory_space=pl.ANY)],
        out_specs=pl.BlockSpec(memory_space=pltpu.VMEM),
        scratch_shapes=[
            pltpu.VMEM((N_Z - 1, ch, n), x.dtype),
            pltpu.SemaphoreType.DMA,
            pltpu.SemaphoreType.DMA((2 * (N_Z - 1),)),
            pltpu.SemaphoreType.DMA((2 * (N_Z - 1),)),
        ],
        compiler_params=pltpu.CompilerParams(
            collective_id=0,
            vmem_limit_bytes=100 * 1024 * 1024,
        ),
    )(x)


# device time: 601882 ns/iter; 1.0014x vs baseline; 1.0014x over previous
import functools

import jax
import jax.numpy as jnp
from jax import lax
from jax.experimental import pallas as pl
from jax.experimental.pallas import tpu as pltpu

N_Z = 4
KC = 8

_MESH = pl.DeviceIdType.MESH


def kernel(x):
    m, n = x.shape
    blk = m // N_Z
    chk = blk // KC
    half = blk // 2

    def body(x_hbm, out_ref, pbuf, sbuf, copy_sem,
             precv, psend, srecv, ssend, ag_send, ag_recv):
        my_x = lax.axis_index("x")
        my_y = lax.axis_index("y")
        my_z = lax.axis_index("z")
        q = 2 * my_x + my_y
        row0 = q * blk
        zr = (my_z + 1) % N_Z
        zl = (my_z + N_Z - 1) % N_Z
        is_mid = jnp.logical_or(my_z == 1, my_z == 2)

        r = 2 * my_x + (my_x ^ my_y)
        rn = (r + 1) % 4
        rp = (r + 3) % 4
        gn = rn ^ (rn // 2)
        gp = rp ^ (rp // 2)
        nx, ny = gn // 2, gn % 2
        px, py = gp // 2, gp % 2

        cp = pltpu.make_async_copy(
            x_hbm.at[pl.ds(row0, blk)], out_ref.at[pl.ds(row0, blk)],
            copy_sem,
        )
        cp.start()
        cp.wait()

        barrier_sem = pltpu.get_barrier_semaphore()
        for dev in ((my_x, my_y, zl), (my_x, my_y, zr),
                    (nx, ny, my_z), (px, py, my_z)):
            pl.semaphore_signal(
                barrier_sem, inc=1, device_id=dev, device_id_type=_MESH,
            )
        pl.semaphore_wait(barrier_sem, 4)

        def pchunk(ref, c):
            return ref.at[pl.ds(c * chk, chk)]

        def ochunk(c):
            return out_ref.at[pl.ds(row0 + c * chk, chk)]

        for c in range(KC):
            @pl.when(my_z == 0)
            def _():
                pltpu.make_async_remote_copy(
                    src_ref=ochunk(c), dst_ref=pchunk(pbuf, c),
                    send_sem=psend.at[c], recv_sem=precv.at[c],
                    device_id=(my_x, my_y, zr), device_id_type=_MESH,
                ).start()

            @pl.when(my_z == 3)
            def _():
                pltpu.make_async_remote_copy(
                    src_ref=ochunk(c), dst_ref=pchunk(sbuf, c),
                    send_sem=ssend.at[c], recv_sem=srecv.at[c],
                    device_id=(my_x, my_y, zl), device_id_type=_MESH,
                ).start()

            @pl.when(is_mid)
            def _():
                pltpu.make_async_remote_copy(
                    src_ref=pchunk(pbuf, c), dst_ref=pchunk(pbuf, c),
                    send_sem=psend.at[c], recv_sem=precv.at[c],
                    device_id=(my_x, my_y, zl), device_id_type=_MESH,
                ).wait_recv()
                pbuf[pl.ds(c * chk, chk), :] = (
                    pbuf[pl.ds(c * chk, chk), :]
                    + out_ref[pl.ds(row0 + c * chk, chk), :]
                )
                pltpu.make_async_remote_copy(
                    src_ref=pchunk(pbuf, c), dst_ref=pchunk(pbuf, c),
                    send_sem=psend.at[c], recv_sem=precv.at[c],
                    device_id=(my_x, my_y, zr), device_id_type=_MESH,
                ).start()

            @pl.when(my_z == 3)
            def _():
                pltpu.make_async_remote_copy(
                    src_ref=pchunk(pbuf, c), dst_ref=pchunk(pbuf, c),
                    send_sem=psend.at[c], recv_sem=precv.at[c],
                    device_id=(my_x, my_y, zl), device_id_type=_MESH,
                ).wait_recv()
                pbuf[pl.ds(c * chk, chk), :] = (
                    pbuf[pl.ds(c * chk, chk), :]
                    + out_ref[pl.ds(row0 + c * chk, chk), :]
                )

            @pl.when(is_mid)
            def _():
                pltpu.make_async_remote_copy(
                    src_ref=pchunk(sbuf, c), dst_ref=pchunk(sbuf, c),
                    send_sem=ssend.at[c], recv_sem=srecv.at[c],
                    device_id=(my_x, my_y, zr), device_id_type=_MESH,
                ).wait_recv()
                sbuf[pl.ds(c * chk, chk), :] = (
                    sbuf[pl.ds(c * chk, chk), :]
                    + out_ref[pl.ds(row0 + c * chk, chk), :]
                )
                pltpu.make_async_remote_copy(
                    src_ref=pchunk(sbuf, c), dst_ref=pchunk(sbuf, c),
                    send_sem=ssend.at[c], recv_sem=srecv.at[c],
                    device_id=(my_x, my_y, zl), device_id_type=_MESH,
                ).start()

            @pl.when(my_z == 0)
            def _():
                pltpu.make_async_remote_copy(
                    src_ref=pchunk(sbuf, c), dst_ref=pchunk(sbuf, c),
                    send_sem=ssend.at[c], recv_sem=srecv.at[c],
                    device_id=(my_x, my_y, zr), device_id_type=_MESH,
                ).wait_recv()

            @pl.when(my_z == 0)
            def _():
                pltpu.make_async_remote_copy(
                    src_ref=ochunk(c), dst_ref=pchunk(pbuf, c),
                    send_sem=psend.at[c], recv_sem=precv.at[c],
                    device_id=(my_x, my_y, zr), device_id_type=_MESH,
                ).wait_send()
                out_ref[pl.ds(row0 + c * chk, chk), :] = (
                    out_ref[pl.ds(row0 + c * chk, chk), :]
                    + sbuf[pl.ds(c * chk, chk), :]
                )

            @pl.when(is_mid)
            def _():
                out_ref[pl.ds(row0 + c * chk, chk), :] = (
                    pbuf[pl.ds(c * chk, chk), :]
                    + sbuf[pl.ds(c * chk, chk), :]
                    - out_ref[pl.ds(row0 + c * chk, chk), :]
                )

            @pl.when(my_z == 3)
            def _():
                pltpu.make_async_remote_copy(
                    src_ref=ochunk(c), dst_ref=pchunk(sbuf, c),
                    send_sem=ssend.at[c], recv_sem=srecv.at[c],
                    device_id=(my_x, my_y, zl), device_id_type=_MESH,
                ).wait_send()
                out_ref[pl.ds(row0 + c * chk, chk), :] = pbuf[
                    pl.ds(c * chk, chk), :
                ]

        @pl.when(is_mid)
        def _():
            for c in range(KC):
                pltpu.make_async_remote_copy(
                    src_ref=pchunk(pbuf, c), dst_ref=pchunk(pbuf, c),
                    send_sem=psend.at[c], recv_sem=precv.at[c],
                    device_id=(my_x, my_y, zr), device_id_type=_MESH,
                ).wait_send()
                pltpu.make_async_remote_copy(
                    src_ref=pchunk(sbuf, c), dst_ref=pchunk(sbuf, c),
                    send_sem=ssend.at[c], recv_sem=srecv.at[c],
                    device_id=(my_x, my_y, zl), device_id_type=_MESH,
                ).wait_send()

        for t in range(N_Z - 1):
            rcw = (r + 4 - t) % 4
            bcw = rcw ^ (rcw // 2)
            rcc = (r + t) % 4
            bcc = rcc ^ (rcc // 2)
            cw = pltpu.make_async_remote_copy(
                src_ref=out_ref.at[pl.ds(bcw * blk, half)],
                dst_ref=out_ref.at[pl.ds(bcw * blk, half)],
                send_sem=ag_send.at[t], recv_sem=ag_recv.at[t],
                device_id=(nx, ny, my_z), device_id_type=_MESH,
            )
            ccw = pltpu.make_async_remote_copy(
                src_ref=out_ref.at[pl.ds(bcc * blk + half, half)],
                dst_ref=out_ref.at[pl.ds(bcc * blk + half, half)],
                send_sem=ag_send.at[N_Z - 1 + t],
                recv_sem=ag_recv.at[N_Z - 1 + t],
                device_id=(px, py, my_z), device_id_type=_MESH,
            )
            cw.start()
            ccw.start()
            cw.wait()
            ccw.wait()

        @functools.partial(
            pl.run_scoped, exit_sem=pltpu.SemaphoreType.REGULAR
        )
        def _(exit_sem):
            for dev in ((my_x, my_y, zl), (my_x, my_y, zr),
                        (nx, ny, my_z), (px, py, my_z)):
                pl.semaphore_signal(
                    exit_sem, inc=1, device_id=dev, device_id_type=_MESH,
                )
            pl.semaphore_wait(exit_sem, 4)

    return pl.pallas_call(
        body,
        out_shape=jax.ShapeDtypeStruct((m, n), x.dtype),
        in_specs=[pl.BlockSpec(memory_space=pl.ANY)],
        out_specs=pl.BlockSpec(memory_space=pltpu.VMEM),
        scratch_shapes=[
            pltpu.VMEM((blk, n), x.dtype),
            pltpu.VMEM((blk, n), x.dtype),
            pltpu.SemaphoreType.DMA,
            pltpu.SemaphoreType.DMA((KC,)),
            pltpu.SemaphoreType.DMA((KC,)),
            pltpu.SemaphoreType.DMA((KC,)),
            pltpu.SemaphoreType.DMA((KC,)),
            pltpu.SemaphoreType.DMA((2 * (N_Z - 1),)),
            pltpu.SemaphoreType.DMA((2 * (N_Z - 1),)),
        ],
        compiler_params=pltpu.CompilerParams(
            collective_id=0,
            vmem_limit_bytes=100 * 1024 * 1024,
        ),
    )(x)


# device time: 337718 ns/iter; 1.7847x vs baseline; 1.7822x over previous
import functools

import jax
import jax.numpy as jnp
from jax import lax
from jax.experimental import pallas as pl
from jax.experimental.pallas import tpu as pltpu

N_Z = 4
KC = 8

_MESH = pl.DeviceIdType.MESH


def kernel(x):
    m, n = x.shape
    blk = m // N_Z
    chk = blk // KC
    half = blk // 2

    def body(x_hbm, out_ref, pbuf, sbuf, copy_sem,
             precv, psend, srecv, ssend, ag_send, ag_recv):
        my_x = lax.axis_index("x")
        my_y = lax.axis_index("y")
        my_z = lax.axis_index("z")
        q = 2 * my_x + my_y
        row0 = q * blk
        zr = (my_z + 1) % N_Z
        zl = (my_z + N_Z - 1) % N_Z
        is_mid = jnp.logical_or(my_z == 1, my_z == 2)

        r = 2 * my_x + (my_x ^ my_y)
        rn = (r + 1) % 4
        rp = (r + 3) % 4
        gn = rn ^ (rn // 2)
        gp = rp ^ (rp // 2)
        nx, ny = gn // 2, gn % 2
        px, py = gp // 2, gp % 2

        cp = pltpu.make_async_copy(
            x_hbm.at[pl.ds(row0, blk)], out_ref.at[pl.ds(row0, blk)],
            copy_sem,
        )
        cp.start()
        cp.wait()

        barrier_sem = pltpu.get_barrier_semaphore()
        for dev in ((my_x, my_y, zl), (my_x, my_y, zr),
                    (nx, ny, my_z), (px, py, my_z)):
            pl.semaphore_signal(
                barrier_sem, inc=1, device_id=dev, device_id_type=_MESH,
            )
        pl.semaphore_wait(barrier_sem, 4)

        def pchunk(ref, c):
            return ref.at[pl.ds(c * chk, chk)]

        def ochunk(c):
            return out_ref.at[pl.ds(row0 + c * chk, chk)]

        S2, S1, S0 = 2, 4, 6
        for c in range(KC + S0):
            if c < KC:
                @pl.when(my_z == 0)
                def _():
                    pltpu.make_async_remote_copy(
                        src_ref=ochunk(c), dst_ref=pchunk(pbuf, c),
                        send_sem=psend.at[c], recv_sem=precv.at[c],
                        device_id=(my_x, my_y, zr), device_id_type=_MESH,
                    ).start()

                @pl.when(my_z == 3)
                def _():
                    pltpu.make_async_remote_copy(
                        src_ref=ochunk(c), dst_ref=pchunk(sbuf, c),
                        send_sem=ssend.at[c], recv_sem=srecv.at[c],
                        device_id=(my_x, my_y, zl), device_id_type=_MESH,
                    ).start()

                @pl.when(is_mid)
                def _():
                    pltpu.make_async_remote_copy(
                        src_ref=pchunk(pbuf, c), dst_ref=pchunk(pbuf, c),
                        send_sem=psend.at[c], recv_sem=precv.at[c],
                        device_id=(my_x, my_y, zl), device_id_type=_MESH,
                    ).wait_recv()
                    pbuf[pl.ds(c * chk, chk), :] = (
                        pbuf[pl.ds(c * chk, chk), :]
                        + out_ref[pl.ds(row0 + c * chk, chk), :]
                    )
                    pltpu.make_async_remote_copy(
                        src_ref=pchunk(pbuf, c), dst_ref=pchunk(pbuf, c),
                        send_sem=psend.at[c], recv_sem=precv.at[c],
                        device_id=(my_x, my_y, zr), device_id_type=_MESH,
                    ).start()

                @pl.when(my_z == 3)
                def _():
                    pltpu.make_async_remote_copy(
                        src_ref=pchunk(pbuf, c), dst_ref=pchunk(pbuf, c),
                        send_sem=psend.at[c], recv_sem=precv.at[c],
                        device_id=(my_x, my_y, zl), device_id_type=_MESH,
                    ).wait_recv()
                    pbuf[pl.ds(c * chk, chk), :] = (
                        pbuf[pl.ds(c * chk, chk), :]
                        + out_ref[pl.ds(row0 + c * chk, chk), :]
                    )
                    pltpu.make_async_remote_copy(
                        src_ref=ochunk(c), dst_ref=pchunk(sbuf, c),
                        send_sem=ssend.at[c], recv_sem=srecv.at[c],
                        device_id=(my_x, my_y, zl), device_id_type=_MESH,
                    ).wait_send()
                    out_ref[pl.ds(row0 + c * chk, chk), :] = pbuf[
                        pl.ds(c * chk, chk), :
                    ]

            c2 = c - S2
            if 0 <= c2 < KC:
                @pl.when(my_z == 2)
                def _():
                    pltpu.make_async_remote_copy(
                        src_ref=pchunk(sbuf, c2), dst_ref=pchunk(sbuf, c2),
                        send_sem=ssend.at[c2], recv_sem=srecv.at[c2],
                        device_id=(my_x, my_y, zr), device_id_type=_MESH,
                    ).wait_recv()
                    sbuf[pl.ds(c2 * chk, chk), :] = (
                        sbuf[pl.ds(c2 * chk, chk), :]
                        + out_ref[pl.ds(row0 + c2 * chk, chk), :]
                    )
                    pltpu.make_async_remote_copy(
                        src_ref=pchunk(sbuf, c2), dst_ref=pchunk(sbuf, c2),
                        send_sem=ssend.at[c2], recv_sem=srecv.at[c2],
                        device_id=(my_x, my_y, zl), device_id_type=_MESH,
                    ).start()
                    out_ref[pl.ds(row0 + c2 * chk, chk), :] = (
                        pbuf[pl.ds(c2 * chk, chk), :]
                        + sbuf[pl.ds(c2 * chk, chk), :]
                        - out_ref[pl.ds(row0 + c2 * chk, chk), :]
                    )

            c1 = c - S1
            if 0 <= c1 < KC:
                @pl.when(my_z == 1)
                def _():
                    pltpu.make_async_remote_copy(
                        src_ref=pchunk(sbuf, c1), dst_ref=pchunk(sbuf, c1),
                        send_sem=ssend.at[c1], recv_sem=srecv.at[c1],
                        device_id=(my_x, my_y, zr), device_id_type=_MESH,
                    ).wait_recv()
                    sbuf[pl.ds(c1 * chk, chk), :] = (
                        sbuf[pl.ds(c1 * chk, chk), :]
                        + out_ref[pl.ds(row0 + c1 * chk, chk), :]
                    )
                    pltpu.make_async_remote_copy(
                        src_ref=pchunk(sbuf, c1), dst_ref=pchunk(sbuf, c1),
                        send_sem=ssend.at[c1], recv_sem=srecv.at[c1],
                        device_id=(my_x, my_y, zl), device_id_type=_MESH,
                    ).start()
                    out_ref[pl.ds(row0 + c1 * chk, chk), :] = (
                        pbuf[pl.ds(c1 * chk, chk), :]
                        + sbuf[pl.ds(c1 * chk, chk), :]
                        - out_ref[pl.ds(row0 + c1 * chk, chk), :]
                    )

            c0 = c - S0
            if 0 <= c0 < KC:
                @pl.when(my_z == 0)
                def _():
                    pltpu.make_async_remote_copy(
                        src_ref=pchunk(sbuf, c0), dst_ref=pchunk(sbuf, c0),
                        send_sem=ssend.at[c0], recv_sem=srecv.at[c0],
                        device_id=(my_x, my_y, zr), device_id_type=_MESH,
                    ).wait_recv()
                    pltpu.make_async_remote_copy(
                        src_ref=ochunk(c0), dst_ref=pchunk(pbuf, c0),
                        send_sem=psend.at[c0], recv_sem=precv.at[c0],
                        device_id=(my_x, my_y, zr), device_id_type=_MESH,
                    ).wait_send()
                    out_ref[pl.ds(row0 + c0 * chk, chk), :] = (
                        out_ref[pl.ds(row0 + c0 * chk, chk), :]
                        + sbuf[pl.ds(c0 * chk, chk), :]
                    )

        @pl.when(is_mid)
        def _():
            for c in range(KC):
                pltpu.make_async_remote_copy(
                    src_ref=pchunk(pbuf, c), dst_ref=pchunk(pbuf, c),
                    send_sem=psend.at[c], recv_sem=precv.at[c],
                    device_id=(my_x, my_y, zr), device_id_type=_MESH,
                ).wait_send()
                pltpu.make_async_remote_copy(
                    src_ref=pchunk(sbuf, c), dst_ref=pchunk(sbuf, c),
                    send_sem=ssend.at[c], recv_sem=srecv.at[c],
                    device_id=(my_x, my_y, zl), device_id_type=_MESH,
                ).wait_send()

        for t in range(N_Z - 1):
            rcw = (r + 4 - t) % 4
            bcw = rcw ^ (rcw // 2)
            rcc = (r + t) % 4
            bcc = rcc ^ (rcc // 2)
            cw = pltpu.make_async_remote_copy(
                src_ref=out_ref.at[pl.ds(bcw * blk, half)],
                dst_ref=out_ref.at[pl.ds(bcw * blk, half)],
                send_sem=ag_send.at[t], recv_sem=ag_recv.at[t],
                device_id=(nx, ny, my_z), device_id_type=_MESH,
            )
            ccw = pltpu.make_async_remote_copy(
                src_ref=out_ref.at[pl.ds(bcc * blk + half, half)],
                dst_ref=out_ref.at[pl.ds(bcc * blk + half, half)],
                send_sem=ag_send.at[N_Z - 1 + t],
                recv_sem=ag_recv.at[N_Z - 1 + t],
                device_id=(px, py, my_z), device_id_type=_MESH,
            )
            cw.start()
            ccw.start()
            cw.wait()
            ccw.wait()

        @functools.partial(
            pl.run_scoped, exit_sem=pltpu.SemaphoreType.REGULAR
        )
        def _(exit_sem):
            for dev in ((my_x, my_y, zl), (my_x, my_y, zr),
                        (nx, ny, my_z), (px, py, my_z)):
                pl.semaphore_signal(
                    exit_sem, inc=1, device_id=dev, device_id_type=_MESH,
                )
            pl.semaphore_wait(exit_sem, 4)

    return pl.pallas_call(
        body,
        out_shape=jax.ShapeDtypeStruct((m, n), x.dtype),
        in_specs=[pl.BlockSpec(memory_space=pl.ANY)],
        out_specs=pl.BlockSpec(memory_space=pltpu.VMEM),
        scratch_shapes=[
            pltpu.VMEM((blk, n), x.dtype),
            pltpu.VMEM((blk, n), x.dtype),
            pltpu.SemaphoreType.DMA,
            pltpu.SemaphoreType.DMA((KC,)),
            pltpu.SemaphoreType.DMA((KC,)),
            pltpu.SemaphoreType.DMA((KC,)),
            pltpu.SemaphoreType.DMA((KC,)),
            pltpu.SemaphoreType.DMA((2 * (N_Z - 1),)),
            pltpu.SemaphoreType.DMA((2 * (N_Z - 1),)),
        ],
        compiler_params=pltpu.CompilerParams(
            collective_id=0,
            vmem_limit_bytes=100 * 1024 * 1024,
        ),
    )(x)
